# Initial kernel scaffold; baseline (speedup 1.0000x reference)
import jax
import jax.numpy as jnp
from jax.experimental import pallas as pl


def kernel(indices, values):
    raise NotImplementedError("write your pallas kernel here")


# speedup vs baseline: 23.3388x; 23.3388x over previous
import functools

import jax
import jax.numpy as jnp
from jax import lax
from jax.experimental import pallas as pl
from jax.experimental.pallas import tpu as pltpu
from jax.experimental.pallas import tpu_sc as plsc

_BATCH = 16
_SEQ = 2048
_OUT = 256
_T = _BATCH * _SEQ * _OUT
_NNZ = 1000000

_NC = 2
_NS = 16
_NW = _NC * _NS

_SUB = 32768
_NSUB = _T // _SUB
_SUB_PER_W = _NSUB // _NW
_E = 4096
_EPAD = _NNZ + _E
_SLEN = _NSUB + 32


def _sc_body(flat_hbm, val_hbm, starts_hbm, out_hbm, stv, stage, fbuf, vbuf):
    cid = lax.axis_index("c")
    sid = lax.axis_index("s")
    wid = sid * _NC + cid

    c0 = wid * _SUB_PER_W
    pltpu.sync_copy(starts_hbm.at[pl.ds(pl.multiple_of(c0, 8), 32)], stv)

    zero16 = jnp.zeros((16,), jnp.float32)

    for k in range(_SUB_PER_W):
        c = c0 + k
        lo = c * _SUB
        hi = lo + _SUB
        s_lo = stv[pl.ds(k, 16)][0]
        s_hi = stv[pl.ds(k + 1, 16)][0]

        def zbody(i, carry):
            for u in range(16):
                stage[pl.ds((i * 16 + u) * 16, 16)] = zero16
            return carry

        lax.fori_loop(0, _SUB // 256, zbody, 0)

        a = (s_lo // 8) * 8
        n = s_hi - a
        nch = (n + _E - 1) // _E

        def ebody(j, carry):
            off = pl.multiple_of((a // 8 + j * (_E // 8)) * 8, 8)
            pltpu.sync_copy(flat_hbm.at[pl.ds(off, _E)], fbuf)
            pltpu.sync_copy(val_hbm.at[pl.ds(off, _E)], vbuf)

            def gbody(g, gc):
                for u in range(4):
                    sl = pl.ds((g * 4 + u) * 16, 16)
                    fv = fbuf[sl]
                    vv = vbuf[sl]
                    m = jnp.logical_and(fv >= lo, fv < hi)
                    plsc.store_scatter(stage, [fv - lo], vv, mask=m)
                return gc

            lax.fori_loop(0, _E // 64, gbody, 0)
            return carry

        lax.fori_loop(0, nch, ebody, 0)

        pltpu.sync_copy(stage, out_hbm.at[pl.ds(pl.multiple_of(lo, 8), _SUB)])


@jax.jit
def _sc_scatter(flat_p, val_p, starts_p):
    mesh = plsc.VectorSubcoreMesh(
        core_axis_name="c", subcore_axis_name="s", num_cores=_NC,
        num_subcores=_NS)
    return pl.kernel(
        _sc_body,
        out_type=jax.ShapeDtypeStruct((_T,), jnp.float32),
        mesh=mesh,
        compiler_params=pltpu.CompilerParams(needs_layout_passes=False),
        scratch_types=[
            pltpu.VMEM((32,), jnp.int32),
            pltpu.VMEM((_SUB,), jnp.float32),
            pltpu.VMEM((_E,), jnp.int32),
            pltpu.VMEM((_E,), jnp.float32),
        ],
    )(flat_p, val_p, starts_p)


def kernel(indices, values):
    idx = indices.astype(jnp.int32)
    flat = idx[:, 0] * (_SEQ * _OUT) + idx[:, 1] * _OUT + idx[:, 2]
    flat_p = jnp.concatenate(
        [flat, jnp.full((_EPAD - _NNZ,), _T, jnp.int32)])
    val_p = jnp.concatenate(
        [values, jnp.zeros((_EPAD - _NNZ,), values.dtype)])
    bounds = jnp.arange(_NSUB + 1, dtype=jnp.int32) * _SUB
    starts = jnp.searchsorted(flat, bounds).astype(jnp.int32)
    starts_p = jnp.concatenate(
        [starts, jnp.full((_SLEN - _NSUB - 1,), _NNZ, jnp.int32)])
    out_flat = _sc_scatter(flat_p, val_p, starts_p)
    return out_flat.reshape(_BATCH, _SEQ, _OUT)


# speedup vs baseline: 23.9934x; 1.0280x over previous
import functools

import jax
import jax.numpy as jnp
from jax import lax
from jax.experimental import pallas as pl
from jax.experimental.pallas import tpu as pltpu
from jax.experimental.pallas import tpu_sc as plsc

_BATCH = 16
_SEQ = 2048
_OUT = 256
_T = _BATCH * _SEQ * _OUT
_NNZ = 1000000

_NC = 2
_NS = 16
_NW = _NC * _NS

_SUB = 32768
_NSUB = _T // _SUB
_SUB_PER_W = _NSUB // _NW
_E = 4096
_SLEN = _NSUB + 32


def _sc_body(flat_hbm, val_hbm, starts_hbm, out_hbm, stv, stage, fbuf, vbuf):
    cid = lax.axis_index("c")
    sid = lax.axis_index("s")
    wid = sid * _NC + cid

    c0 = wid * _SUB_PER_W
    pltpu.sync_copy(starts_hbm.at[pl.ds(pl.multiple_of(c0, 8), 32)], stv)

    zero16 = jnp.zeros((16,), jnp.float32)

    for k in range(_SUB_PER_W):
        c = c0 + k
        lo = c * _SUB
        hi = lo + _SUB
        s_lo = stv[pl.ds(k, 16)][0]
        s_hi = stv[pl.ds(k + 1, 16)][0]

        def zbody(i, carry):
            for u in range(16):
                stage[pl.ds((i * 16 + u) * 16, 16)] = zero16
            return carry

        lax.fori_loop(0, _SUB // 256, zbody, 0)

        a = (s_lo // 8) * 8
        n = s_hi - a
        nch = (n + _E - 1) // _E

        def ebody(j, carry):
            off = jnp.minimum(a + j * _E, _NNZ - _E)
            off = pl.multiple_of((off // 8) * 8, 8)
            pltpu.sync_copy(flat_hbm.at[pl.ds(off, _E)], fbuf)
            pltpu.sync_copy(val_hbm.at[pl.ds(off, _E)], vbuf)

            def gbody(g, gc):
                for u in range(4):
                    sl = pl.ds((g * 4 + u) * 16, 16)
                    fv = fbuf[sl]
                    vv = vbuf[sl]
                    m = jnp.logical_and(fv >= lo, fv < hi)
                    plsc.store_scatter(stage, [fv - lo], vv, mask=m)
                return gc

            lax.fori_loop(0, _E // 64, gbody, 0)
            return carry

        lax.fori_loop(0, nch, ebody, 0)

        pltpu.sync_copy(stage, out_hbm.at[pl.ds(pl.multiple_of(lo, 8), _SUB)])


@jax.jit
def _sc_scatter(flat_p, val_p, starts_p):
    mesh = plsc.VectorSubcoreMesh(
        core_axis_name="c", subcore_axis_name="s", num_cores=_NC,
        num_subcores=_NS)
    return pl.kernel(
        _sc_body,
        out_type=jax.ShapeDtypeStruct((_T,), jnp.float32),
        mesh=mesh,
        compiler_params=pltpu.CompilerParams(needs_layout_passes=False),
        scratch_types=[
            pltpu.VMEM((32,), jnp.int32),
            pltpu.VMEM((_SUB,), jnp.float32),
            pltpu.VMEM((_E,), jnp.int32),
            pltpu.VMEM((_E,), jnp.float32),
        ],
    )(flat_p, val_p, starts_p)


def kernel(indices, values):
    idx = indices.astype(jnp.int32)
    flat = idx[:, 0] * (_SEQ * _OUT) + idx[:, 1] * _OUT + idx[:, 2]
    bounds = jnp.arange(_NSUB + 1, dtype=jnp.int32) * _SUB
    starts = jnp.searchsorted(
        flat, bounds, method="scan_unrolled").astype(jnp.int32)
    starts_p = jnp.concatenate(
        [starts, jnp.full((_SLEN - _NSUB - 1,), _NNZ, jnp.int32)])
    out_flat = _sc_scatter(flat, values, starts_p)
    return out_flat.reshape(_BATCH, _SEQ, _OUT)


# speedup vs baseline: 28.7252x; 1.1972x over previous
import functools

import jax
import jax.numpy as jnp
from jax import lax
from jax.experimental import pallas as pl
from jax.experimental.pallas import tpu as pltpu
from jax.experimental.pallas import tpu_sc as plsc

_BATCH = 16
_SEQ = 2048
_OUT = 256
_T = _BATCH * _SEQ * _OUT
_NNZ = 1000000

_NC = 2
_NS = 16
_NW = _NC * _NS

_SUB = 32768
_NSUB = _T // _SUB
_SUB_PER_W = _NSUB // _NW
_E = 4096


def _sc_body(flat_hbm, val_hbm, out_hbm, gbuf, gsem, stage, fbuf, vbuf):
    cid = lax.axis_index("c")
    sid = lax.axis_index("s")
    wid = sid * _NC + cid
    c0 = wid * _SUB_PER_W

    lanes = lax.iota(jnp.int32, 16)
    q = (c0 + jnp.minimum(lanes, _SUB_PER_W)) * _SUB
    blo = jnp.zeros((16,), jnp.int32)
    bhi = jnp.full((16,), _NNZ, jnp.int32)
    for _ in range(20):
        upd = blo < bhi
        mid = jnp.minimum((blo + bhi) >> 1, _NNZ - 1)
        pltpu.async_copy(flat_hbm.at[mid], gbuf, gsem).wait()
        lt = gbuf[...] < q
        blo = jnp.where(jnp.logical_and(upd, lt), mid + 1, blo)
        bhi = jnp.where(jnp.logical_and(upd, jnp.logical_not(lt)), mid, bhi)

    zero16 = jnp.zeros((16,), jnp.float32)

    for k in range(_SUB_PER_W):
        c = c0 + k
        lo = c * _SUB
        hi = lo + _SUB
        s_lo = blo[k]
        s_hi = blo[k + 1]

        def zbody(i, carry):
            for u in range(16):
                stage[pl.ds((i * 16 + u) * 16, 16)] = zero16
            return carry

        lax.fori_loop(0, _SUB // 256, zbody, 0)

        a = (s_lo // 8) * 8
        n = s_hi - a
        nch = (n + _E - 1) // _E

        def ebody(j, carry):
            off = jnp.minimum(a + j * _E, _NNZ - _E)
            off = pl.multiple_of((off // 8) * 8, 8)
            pltpu.sync_copy(flat_hbm.at[pl.ds(off, _E)], fbuf)
            pltpu.sync_copy(val_hbm.at[pl.ds(off, _E)], vbuf)

            def gbody(g, gc):
                for u in range(4):
                    sl = pl.ds((g * 4 + u) * 16, 16)
                    fv = fbuf[sl]
                    vv = vbuf[sl]
                    m = jnp.logical_and(fv >= lo, fv < hi)
                    plsc.store_scatter(stage, [fv - lo], vv, mask=m)
                return gc

            lax.fori_loop(0, _E // 64, gbody, 0)
            return carry

        lax.fori_loop(0, nch, ebody, 0)

        pltpu.sync_copy(stage, out_hbm.at[pl.ds(pl.multiple_of(lo, 8), _SUB)])


@jax.jit
def _sc_scatter(flat_p, val_p):
    mesh = plsc.VectorSubcoreMesh(
        core_axis_name="c", subcore_axis_name="s", num_cores=_NC,
        num_subcores=_NS)
    return pl.kernel(
        _sc_body,
        out_type=jax.ShapeDtypeStruct((_T,), jnp.float32),
        mesh=mesh,
        compiler_params=pltpu.CompilerParams(needs_layout_passes=False),
        scratch_types=[
            pltpu.VMEM((16,), jnp.int32),
            pltpu.SemaphoreType.DMA,
            pltpu.VMEM((_SUB,), jnp.float32),
            pltpu.VMEM((_E,), jnp.int32),
            pltpu.VMEM((_E,), jnp.float32),
        ],
    )(flat_p, val_p)


def kernel(indices, values):
    idx = indices.astype(jnp.int32)
    flat = idx[:, 0] * (_SEQ * _OUT) + idx[:, 1] * _OUT + idx[:, 2]
    out_flat = _sc_scatter(flat, values)
    return out_flat.reshape(_BATCH, _SEQ, _OUT)


# speedup vs baseline: 28.8219x; 1.0034x over previous
import functools

import jax
import jax.numpy as jnp
from jax import lax
from jax.experimental import pallas as pl
from jax.experimental.pallas import tpu as pltpu
from jax.experimental.pallas import tpu_sc as plsc

_BATCH = 16
_SEQ = 2048
_OUT = 256
_T = _BATCH * _SEQ * _OUT
_NNZ = 1000000

_NC = 2
_NS = 16
_NW = _NC * _NS

_SUB = 32768
_NSUB = _T // _SUB
_SUB_PER_W = _NSUB // _NW
_E = 4096


def _sc_body(flat_hbm, val_hbm, out_hbm, gbuf, gsem, stage, fbuf, vbuf):
    cid = lax.axis_index("c")
    sid = lax.axis_index("s")
    wid = sid * _NC + cid
    c0 = wid * _SUB_PER_W

    lanes = lax.iota(jnp.int32, 16)
    q = (c0 + jnp.minimum(lanes, _SUB_PER_W)) * _SUB
    blo = jnp.zeros((16,), jnp.int32)
    bhi = jnp.full((16,), _NNZ, jnp.int32)
    for _ in range(20):
        upd = blo < bhi
        mid = jnp.minimum((blo + bhi) >> 1, _NNZ - 1)
        pltpu.async_copy(flat_hbm.at[mid], gbuf, gsem).wait()
        lt = gbuf[...] < q
        blo = jnp.where(jnp.logical_and(upd, lt), mid + 1, blo)
        bhi = jnp.where(jnp.logical_and(upd, jnp.logical_not(lt)), mid, bhi)

    zero16 = jnp.zeros((16,), jnp.float32)

    for k in range(_SUB_PER_W):
        c = c0 + k
        lo = c * _SUB
        hi = lo + _SUB
        s_lo = blo[k]
        s_hi = blo[k + 1]

        def zbody(i, carry):
            for u in range(16):
                stage[pl.ds((i * 16 + u) * 16, 16)] = zero16
            return carry

        lax.fori_loop(0, _SUB // 256, zbody, 0)

        a = (s_lo // 8) * 8
        n = s_hi - a
        nch = (n + _E - 1) // _E

        def ebody(j, carry):
            off = jnp.minimum(a + j * _E, _NNZ - _E)
            off = pl.multiple_of((off // 8) * 8, 8)
            pltpu.sync_copy(flat_hbm.at[pl.ds(off, _E)], fbuf)
            pltpu.sync_copy(val_hbm.at[pl.ds(off, _E)], vbuf)

            def gbody(g, gc):
                for u in range(4):
                    sl = pl.ds((g * 4 + u) * 16, 16)
                    fv = fbuf[sl]
                    vv = vbuf[sl]
                    m = jnp.logical_and(fv >= lo, fv < hi)
                    plsc.store_scatter(stage, [fv - lo], vv, mask=m)
                return gc

            lax.fori_loop(0, _E // 64, gbody, 0)
            return carry

        lax.fori_loop(0, nch, ebody, 0)

        pltpu.sync_copy(stage, out_hbm.at[pl.ds(pl.multiple_of(lo, 8), _SUB)])


@jax.jit
def _sc_scatter(flat_p, val_p):
    mesh = plsc.VectorSubcoreMesh(
        core_axis_name="c", subcore_axis_name="s", num_cores=_NC,
        num_subcores=_NS)
    return pl.kernel(
        _sc_body,
        out_type=jax.ShapeDtypeStruct((_T,), jnp.float32),
        mesh=mesh,
        compiler_params=pltpu.CompilerParams(needs_layout_passes=False),
        scratch_types=[
            pltpu.VMEM((16,), jnp.int32),
            pltpu.SemaphoreType.DMA,
            pltpu.VMEM((_SUB,), jnp.float32),
            pltpu.VMEM((_E,), jnp.int32),
            pltpu.VMEM((_E,), jnp.float32),
        ],
    )(flat_p, val_p)


def kernel(indices, values):
    w = jnp.array([_SEQ * _OUT, _OUT, 1], dtype=jnp.float32)
    flat = jnp.dot(indices.astype(jnp.float32), w).astype(jnp.int32)
    out_flat = _sc_scatter(flat, values)
    return out_flat.reshape(_BATCH, _SEQ, _OUT)
